# Initial kernel scaffold; baseline (speedup 1.0000x reference)
import jax
import jax.numpy as jnp
from jax.experimental import pallas as pl


def kernel(query, key, value, W_q_down, W_k_down, W_v_down, u_q, v_q, u_k, v_k, W_o):
    raise NotImplementedError("write your pallas kernel here")


# speedup vs baseline: 31.8820x; 31.8820x over previous
import math

import jax
import jax.numpy as jnp
from jax.experimental import pallas as pl

L = 2048
DM = 1024
DK = 64
RANK = 32
H = 12
KMAX = 32

BQ = 256
NCH = 16
CH = 128

_NEG = -1e9
_SCALE = 1.0 / 8.0


def _prep_kernel(q_ref, k_ref, v_ref, wq_ref, wk_ref, wv_ref,
                 uq_ref, vq_ref, uk_ref, vk_ref,
                 qproj_ref, kd_ref, vd_ref):
    q_down = jnp.dot(q_ref[...], wq_ref[...], preferred_element_type=jnp.float32)
    kd_ref[...] = jnp.dot(k_ref[...], wk_ref[...], preferred_element_type=jnp.float32)
    vd_ref[...] = jnp.dot(v_ref[...], wv_ref[...], preferred_element_type=jnp.float32)
    for h in range(H):
        w_uq = jnp.dot(uq_ref[h], vq_ref[h], preferred_element_type=jnp.float32)
        w_uk = jnp.dot(uk_ref[h], vk_ref[h], preferred_element_type=jnp.float32)
        wabs_t = jax.lax.dot_general(w_uq, w_uk, (((0,), (0,)), ((), ())),
                                     preferred_element_type=jnp.float32)
        qproj_ref[h] = jnp.dot(q_down, wabs_t, preferred_element_type=jnp.float32)


def _attn_kernel(qp_ref, kd_ref, vd_ref, wo_ref, out_ref):
    h = pl.program_id(1)
    qp = qp_ref[0]
    kd = kd_ref[...]
    vd = vd_ref[...]

    q_pm = jnp.where(qp > 0, 1.0, -1.0).astype(jnp.bfloat16)
    k_pm = jnp.where(kd > 0, 1.0, -1.0).astype(jnp.bfloat16)
    s = jax.lax.dot_general(q_pm, k_pm, (((1,), (1,)), ((), ())),
                            preferred_element_type=jnp.float32)
    match = (s >= DK - 0.5).astype(jnp.float32)

    m2 = match.astype(jnp.bfloat16).reshape(BQ * NCH, CH)
    row = jax.lax.broadcasted_iota(jnp.int32, (CH, CH), 0)
    col = jax.lax.broadcasted_iota(jnp.int32, (CH, CH), 1)
    upper_incl = (row <= col).astype(jnp.bfloat16)
    pre = jnp.dot(m2, upper_incl, preferred_element_type=jnp.float32)
    pre3 = pre.reshape(BQ, NCH, CH)

    tot = jnp.sum(match.reshape(BQ, NCH, CH), axis=2)
    crow = jax.lax.broadcasted_iota(jnp.int32, (NCH, NCH), 0)
    ccol = jax.lax.broadcasted_iota(jnp.int32, (NCH, NCH), 1)
    strict = (crow < ccol).astype(jnp.bfloat16)
    off = jnp.dot(tot.astype(jnp.bfloat16), strict,
                  preferred_element_type=jnp.float32)

    rank3 = pre3 + off[:, :, None]
    keep3 = jnp.where((match.reshape(BQ, NCH, CH) > 0.5)
                      & (rank3 <= KMAX + 0.5), 1.0, 0.0)
    keep = keep3.reshape(BQ, L)

    scores = jax.lax.dot_general(qp, kd, (((1,), (1,)), ((), ())),
                                 preferred_element_type=jnp.float32) * _SCALE
    scores = jnp.where(keep > 0.5, scores, _NEG)
    mx = jnp.max(scores, axis=1, keepdims=True)
    e = jnp.exp(scores - mx)
    w = e / jnp.sum(e, axis=1, keepdims=True) * keep

    part = jnp.dot(w, vd, preferred_element_type=jnp.float32)
    contrib = jnp.dot(part, wo_ref[0], preferred_element_type=jnp.float32)

    @pl.when(h == 0)
    def _():
        out_ref[...] = contrib

    @pl.when(h > 0)
    def _():
        out_ref[...] += contrib


def kernel(query, key, value, W_q_down, W_k_down, W_v_down,
           u_q, v_q, u_k, v_k, W_o):
    q2 = query.reshape(L, DM)
    k2 = key.reshape(L, DM)
    v2 = value.reshape(L, DM)

    qproj, kd, vd = pl.pallas_call(
        _prep_kernel,
        out_shape=(
            jax.ShapeDtypeStruct((H, L, DK), jnp.float32),
            jax.ShapeDtypeStruct((L, DK), jnp.float32),
            jax.ShapeDtypeStruct((L, DK), jnp.float32),
        ),
    )(q2, k2, v2, W_q_down, W_k_down, W_v_down, u_q, v_q, u_k, v_k)

    wo3 = W_o.reshape(H, DK, DM)

    out = pl.pallas_call(
        _attn_kernel,
        grid=(L // BQ, H),
        in_specs=[
            pl.BlockSpec((1, BQ, DK), lambda qb, h: (h, qb, 0)),
            pl.BlockSpec((L, DK), lambda qb, h: (0, 0)),
            pl.BlockSpec((L, DK), lambda qb, h: (0, 0)),
            pl.BlockSpec((1, DK, DM), lambda qb, h: (h, 0, 0)),
        ],
        out_specs=pl.BlockSpec((BQ, DM), lambda qb, h: (qb, 0)),
        out_shape=jax.ShapeDtypeStruct((L, DM), jnp.float32),
    )(qproj, kd, vd, wo3)

    return out.reshape(1, L, DM)


# speedup vs baseline: 101.2549x; 3.1759x over previous
import math

import jax
import jax.numpy as jnp
from jax.experimental import pallas as pl

L = 2048
DM = 1024
DK = 64
RANK = 32
H = 12
KMAX = 32

BQ = 256
NCH = 16
CH = 128

_NEG = -1e9
_SCALE = 1.0 / 8.0


def _prep_kernel(q_ref, k_ref, v_ref, wq_ref, wk_ref, wv_ref,
                 uq_ref, vq_ref, uk_ref, vk_ref,
                 qproj_ref, qpm_ref, kd_ref, kpm_ref, vd_ref):
    q_down = jnp.dot(q_ref[...], wq_ref[...], preferred_element_type=jnp.float32)
    kd = jnp.dot(k_ref[...], wk_ref[...], preferred_element_type=jnp.float32)
    kd_ref[...] = kd
    kpm_ref[...] = jnp.where(kd > 0, 1.0, -1.0).astype(jnp.bfloat16)
    vd_ref[...] = jnp.dot(v_ref[...], wv_ref[...], preferred_element_type=jnp.float32)
    for h in range(H):
        w_uq = jnp.dot(uq_ref[h], vq_ref[h], preferred_element_type=jnp.float32)
        w_uk = jnp.dot(uk_ref[h], vk_ref[h], preferred_element_type=jnp.float32)
        wabs_t = jax.lax.dot_general(w_uq, w_uk, (((0,), (0,)), ((), ())),
                                     preferred_element_type=jnp.float32)
        qp = jnp.dot(q_down, wabs_t, preferred_element_type=jnp.float32)
        qproj_ref[h] = qp
        qpm_ref[h] = jnp.where(qp > 0, 1.0, -1.0).astype(jnp.bfloat16)


def _attn_kernel(qp_ref, qpm_ref, kd_ref, kpm_ref, vd_ref, wo_ref, out_ref):
    h = pl.program_id(1)
    s = jax.lax.dot_general(qpm_ref[0], kpm_ref[...], (((1,), (1,)), ((), ())),
                            preferred_element_type=jnp.float32)
    any_match = jnp.max(s) >= DK - 0.5

    @pl.when(h == 0)
    def _():
        out_ref[...] = jnp.zeros_like(out_ref)

    @pl.when(any_match)
    def _():
        qp = qp_ref[0]
        kd = kd_ref[...]
        match = (s >= DK - 0.5).astype(jnp.float32)

        m2 = match.astype(jnp.bfloat16).reshape(BQ * NCH, CH)
        row = jax.lax.broadcasted_iota(jnp.int32, (CH, CH), 0)
        col = jax.lax.broadcasted_iota(jnp.int32, (CH, CH), 1)
        upper_incl = (row <= col).astype(jnp.bfloat16)
        pre = jnp.dot(m2, upper_incl, preferred_element_type=jnp.float32)
        pre3 = pre.reshape(BQ, NCH, CH)

        tot = jnp.sum(match.reshape(BQ, NCH, CH), axis=2)
        crow = jax.lax.broadcasted_iota(jnp.int32, (NCH, NCH), 0)
        ccol = jax.lax.broadcasted_iota(jnp.int32, (NCH, NCH), 1)
        strict = (crow < ccol).astype(jnp.bfloat16)
        off = jnp.dot(tot.astype(jnp.bfloat16), strict,
                      preferred_element_type=jnp.float32)

        rank3 = pre3 + off[:, :, None]
        keep3 = jnp.where((match.reshape(BQ, NCH, CH) > 0.5)
                          & (rank3 <= KMAX + 0.5), 1.0, 0.0)
        keep = keep3.reshape(BQ, L)

        scores = jax.lax.dot_general(qp, kd, (((1,), (1,)), ((), ())),
                                     preferred_element_type=jnp.float32) * _SCALE
        scores = jnp.where(keep > 0.5, scores, _NEG)
        mx = jnp.max(scores, axis=1, keepdims=True)
        e = jnp.exp(scores - mx)
        w = e / jnp.sum(e, axis=1, keepdims=True) * keep

        part = jnp.dot(w, vd_ref[...], preferred_element_type=jnp.float32)
        out_ref[...] += jnp.dot(part, wo_ref[0],
                                preferred_element_type=jnp.float32)


def kernel(query, key, value, W_q_down, W_k_down, W_v_down,
           u_q, v_q, u_k, v_k, W_o):
    q2 = query.reshape(L, DM)
    k2 = key.reshape(L, DM)
    v2 = value.reshape(L, DM)

    qproj, qpm, kd, kpm, vd = pl.pallas_call(
        _prep_kernel,
        out_shape=(
            jax.ShapeDtypeStruct((H, L, DK), jnp.float32),
            jax.ShapeDtypeStruct((H, L, DK), jnp.bfloat16),
            jax.ShapeDtypeStruct((L, DK), jnp.float32),
            jax.ShapeDtypeStruct((L, DK), jnp.bfloat16),
            jax.ShapeDtypeStruct((L, DK), jnp.float32),
        ),
    )(q2, k2, v2, W_q_down, W_k_down, W_v_down, u_q, v_q, u_k, v_k)

    wo3 = W_o.reshape(H, DK, DM)

    out = pl.pallas_call(
        _attn_kernel,
        grid=(L // BQ, H),
        in_specs=[
            pl.BlockSpec((1, BQ, DK), lambda qb, h: (h, qb, 0)),
            pl.BlockSpec((1, BQ, DK), lambda qb, h: (h, qb, 0)),
            pl.BlockSpec((L, DK), lambda qb, h: (0, 0)),
            pl.BlockSpec((L, DK), lambda qb, h: (0, 0)),
            pl.BlockSpec((L, DK), lambda qb, h: (0, 0)),
            pl.BlockSpec((1, DK, DM), lambda qb, h: (h, 0, 0)),
        ],
        out_specs=pl.BlockSpec((BQ, DM), lambda qb, h: (qb, 0)),
        out_shape=jax.ShapeDtypeStruct((L, DM), jnp.float32),
    )(qproj, qpm, kd, kpm, vd, wo3)

    return out.reshape(1, L, DM)
